# Initial kernel scaffold; baseline (speedup 1.0000x reference)
import jax
import jax.numpy as jnp
from jax.experimental import pallas as pl


def kernel(feat, edge_index, edge_weight, beta, eps):
    raise NotImplementedError("write your pallas kernel here")


# speedup vs baseline: 19.7335x; 19.7335x over previous
import functools

import jax
import jax.numpy as jnp
from jax import lax
from jax.experimental import pallas as pl
from jax.experimental.pallas import tpu as pltpu
from jax.experimental.pallas import tpu_sc as plsc

N = 10000
E = 320000
D = 128

NC = 2
NS = 16
NW = NC * NS

C = 128
K = 80
EPW = K * C
EP = NW * EPW
NP = 10240
RPT = NP // NS
NB = 2

_mesh = plsc.VectorSubcoreMesh(core_axis_name="c", subcore_axis_name="s")


def _lane_bcast(v, i):
    return jax.lax.gather(
        v,
        jnp.full((16, 1), i, jnp.int32),
        jax.lax.GatherDimensionNumbers(
            offset_dims=(), collapsed_slice_dims=(0,), start_index_map=(0,)),
        (1,),
        mode=jax.lax.GatherScatterMode.PROMISE_IN_BOUNDS,
    )


@functools.partial(
    pl.kernel,
    out_type=jax.ShapeDtypeStruct((NC, NP), jnp.float32),
    mesh=_mesh,
    scratch_types=[
        pltpu.VMEM((EPW,), jnp.float32),
        pltpu.VMEM((K, C), jnp.int32),
        pltpu.VMEM((16,), jnp.float32),
        pltpu.VMEM((RPT,), jnp.float32),
        pltpu.VMEM_SHARED((NP,), jnp.float32),
    ],
)
def _denom(ew2, src3, beta16, den_out, ew_v, idx_v, bvec, zsl, den_sh):
    c = lax.axis_index("c")
    s = lax.axis_index("s")
    w = c * NS + s

    pltpu.sync_copy(ew2.at[w], ew_v)
    pltpu.sync_copy(src3.at[w], idx_v)
    pltpu.sync_copy(beta16, bvec)
    bv = bvec[...]

    def _exp_body(i, carry):
        sl = pl.ds(i * 16, 16)
        ew_v[sl] = jnp.exp(bv * ew_v[sl])
        return carry
    lax.fori_loop(0, EPW // 16, _exp_body, 0)

    def _zero_body(i, carry):
        zsl[pl.ds(i * 16, 16)] = jnp.zeros((16,), jnp.float32)
        return carry
    lax.fori_loop(0, RPT // 16, _zero_body, 0)
    pltpu.sync_copy(zsl, den_sh.at[pl.ds(s * RPT, RPT)])
    plsc.subcore_barrier()

    def _scat_body(k, carry):
        pltpu.sync_copy(ew_v.at[pl.ds(k * C, C)], den_sh.at[idx_v.at[k]],
                        add=True)
        return carry
    lax.fori_loop(0, K, _scat_body, 0)
    plsc.subcore_barrier()

    pltpu.sync_copy(den_sh.at[pl.ds(s * RPT, RPT)], zsl)
    pltpu.sync_copy(zsl, den_out.at[c, pl.ds(s * RPT, RPT)])


NR = 4


@functools.partial(
    pl.kernel,
    out_type=jax.ShapeDtypeStruct((NC, NP, D), jnp.float32),
    mesh=_mesh,
    scratch_types=[
        pltpu.VMEM((NR, C), jnp.int32),
        pltpu.VMEM((NR, C), jnp.int32),
        pltpu.VMEM((NR, C), jnp.float32),
        pltpu.VMEM((16,), jnp.float32),
        [pltpu.VMEM((C, D), jnp.float32) for _ in range(NB)],
        pltpu.VMEM_SHARED((NP, D), jnp.float32),
        [pltpu.SemaphoreType.DMA for _ in range(NR)],
        [pltpu.SemaphoreType.DMA for _ in range(NB)],
        [pltpu.SemaphoreType.DMA for _ in range(NB)],
    ],
)
def _aggregate(g_hbm, ew2, src3, dst3, beta16, h_out,
               sidx, didx, ewx, bvec, rows, h_sh, isems, gsems, ssems):
    c = lax.axis_index("c")
    s = lax.axis_index("s")
    w = c * NS + s

    pltpu.sync_copy(beta16, bvec)
    bv = bvec[...]

    def _prefetch(slot, j):
        pltpu.async_copy(src3.at[w, j], sidx.at[slot], isems[slot])
        pltpu.async_copy(dst3.at[w, j], didx.at[slot], isems[slot])
        pltpu.async_copy(ew2.at[w, pl.ds(j * C, C)], ewx.at[slot], isems[slot])

    def _wait_prefetch(slot, j):
        pltpu.make_async_copy(src3.at[w, j], sidx.at[slot], isems[slot]).wait()
        pltpu.make_async_copy(dst3.at[w, j], didx.at[slot], isems[slot]).wait()
        pltpu.make_async_copy(
            ew2.at[w, pl.ds(j * C, C)], ewx.at[slot], isems[slot]).wait()

    def _zrow(i, carry):
        for q in range(D // 16):
            rows[0][i, pl.ds(q * 16, 16)] = jnp.zeros((16,), jnp.float32)
        return carry
    lax.fori_loop(0, C, _zrow, 0)
    for m in range(RPT // C):
        pltpu.sync_copy(rows[0], h_sh.at[pl.ds(s * RPT + m * C, C)])
    plsc.subcore_barrier()

    _prefetch(0, 0)
    _prefetch(1, 1)
    _wait_prefetch(0, 0)
    pltpu.async_copy(g_hbm.at[sidx.at[0]], rows[0], gsems[0])

    def _step(it, carry):
        jo = it * NR
        for bi in range(NR):
            j = jo + bi
            b = bi % NB
            sn = (bi + 2) % NR

            @pl.when(j + 2 < K)
            def _pf():
                _prefetch(sn, j + 2)

            pltpu.make_async_copy(
                g_hbm.at[sidx.at[bi]], rows[b], gsems[b]).wait()

            for q in range(D // 16):
                sl = pl.ds(q * 16, 16)
                ewx[bi, sl] = jnp.exp(bv * ewx[bi, sl])

            def _scale(ii, carry2):
                p16 = ewx[bi, pl.ds(ii * 16, 16)]
                for i in range(16):
                    pv = _lane_bcast(p16, i)
                    r = ii * 16 + i
                    for q in range(D // 16):
                        sl = pl.ds(q * 16, 16)
                        rows[b][r, sl] = rows[b][r, sl] * pv
                return carry2
            lax.fori_loop(0, C // 16, _scale, 0)

            pltpu.async_copy(rows[b], h_sh.at[didx.at[bi]], ssems[b], add=True)

            jn = j + 1
            bj = (bi + 1) % NR
            bn = (bi + 1) % NB
            @pl.when(jn < K)
            def _issue():
                @pl.when(jn >= NB)
                def _drain():
                    pltpu.make_async_copy(
                        rows[bn], h_sh.at[didx.at[bj]], ssems[bn]).wait()
                _wait_prefetch(bj, jn)
                pltpu.async_copy(g_hbm.at[sidx.at[bj]], rows[bn], gsems[bn])
        return carry
    lax.fori_loop(0, K // NR, _step, 0)
    for i in range(NB):
        jd = K - NB + i
        pltpu.make_async_copy(
            rows[jd % NB], h_sh.at[didx.at[jd % NR]], ssems[jd % NB]).wait()
    plsc.subcore_barrier()

    for m in range(RPT // C):
        r0 = s * RPT + m * C
        pltpu.sync_copy(h_sh.at[pl.ds(r0, C)], rows[0])
        pltpu.sync_copy(rows[0], h_out.at[c, pl.ds(r0, C), :])


def _scale_rows_body(feat_ref, d0_ref, d1_ref, g_ref):
    f = feat_ref[...]
    nr = jnp.sqrt(jnp.sum(f * f, axis=1, keepdims=True))
    nh = f / jnp.maximum(nr, 1e-12)
    d = d0_ref[...] + d1_ref[...]
    g_ref[...] = nh / jnp.maximum(d, 1e-30)


def _combine_body(feat_ref, h0_ref, h1_ref, sc_ref, o_ref):
    o_ref[...] = sc_ref[0, 0] * feat_ref[...] + h0_ref[0] + h1_ref[0]


_BR = 1024
_BRO = 1000


def kernel(feat, edge_index, edge_weight, beta, eps):
    src = edge_index[0]
    dst = edge_index[1]
    ew = edge_weight.reshape(E)

    pad = EP - E
    pad_idx = (N + (jnp.arange(pad, dtype=jnp.int32) % (NP - N))).astype(jnp.int32)
    src_p = jnp.concatenate([src, pad_idx]).reshape(NW, K, C)
    dst_p = jnp.concatenate([dst, pad_idx]).reshape(NW, K, C)
    ew_p = jnp.concatenate([ew, jnp.zeros((pad,), jnp.float32)]).reshape(NW, EPW)
    feat_p = jnp.concatenate(
        [feat, jnp.zeros((NP - N, D), jnp.float32)], axis=0)
    beta16 = jnp.broadcast_to(beta, (16,)).astype(jnp.float32)
    scale = (1.0 + eps).reshape(1, 1).astype(jnp.float32)

    denoms = _denom(ew_p, src_p, beta16)

    g = pl.pallas_call(
        _scale_rows_body,
        grid=(NP // _BR,),
        in_specs=[
            pl.BlockSpec((_BR, D), lambda i: (i, 0)),
            pl.BlockSpec((_BR, 1), lambda i: (i, 0)),
            pl.BlockSpec((_BR, 1), lambda i: (i, 0)),
        ],
        out_specs=pl.BlockSpec((_BR, D), lambda i: (i, 0)),
        out_shape=jax.ShapeDtypeStruct((NP, D), jnp.float32),
    )(feat_p, denoms[0].reshape(NP, 1), denoms[1].reshape(NP, 1))

    h_part = _aggregate(g, ew_p, src_p, dst_p, beta16)

    rst = pl.pallas_call(
        _combine_body,
        grid=(N // _BRO,),
        in_specs=[
            pl.BlockSpec((_BRO, D), lambda i: (i, 0)),
            pl.BlockSpec((1, _BRO, D), lambda i: (0, i, 0)),
            pl.BlockSpec((1, _BRO, D), lambda i: (1, i, 0)),
            pl.BlockSpec((1, 1), lambda i: (0, 0)),
        ],
        out_specs=pl.BlockSpec((_BRO, D), lambda i: (i, 0)),
        out_shape=jax.ShapeDtypeStruct((N, D), jnp.float32),
    )(feat, h_part, h_part, scale)

    return rst
